# Initial kernel scaffold; baseline (speedup 1.0000x reference)
import jax
import jax.numpy as jnp
from jax.experimental import pallas as pl


def kernel(x, token_table, pos_table):
    raise NotImplementedError("write your pallas kernel here")


# speedup vs baseline: 2.5631x; 2.5631x over previous
import functools

import jax
import jax.numpy as jnp
from jax import lax
from jax.experimental import pallas as pl
from jax.experimental.pallas import tpu as pltpu
from jax.experimental.pallas import tpu_sc as plsc

T = 200
D = 64
B = 4096

NC = 2
NS = 16
NW = NC * NS
LOOK = B * T
PER_W = LOOK // NW
CHUNK = 128
NCHUNK = PER_W // CHUNK
LANES = 16


def _body(x_hbm, tok_hbm, pos_hbm, out_hbm, idx_v, pos2_v, buf0, buf1,
          sem0, sem1):
    wid = lax.axis_index("s") * NC + lax.axis_index("c")
    base_chunk = wid * NCHUNK
    row_base = wid * PER_W

    pltpu.sync_copy(x_hbm.at[pl.ds(base_chunk, NCHUNK), :], idx_v)
    pltpu.sync_copy(pos_hbm, pos2_v.at[pl.ds(0, T), :])
    pltpu.sync_copy(pos_hbm, pos2_v.at[pl.ds(T, T), :])

    def start(k, buf, sem):
        pltpu.make_async_copy(tok_hbm.at[idx_v.at[k]], buf, sem).start()

    def finish(k, buf, sem):
        pltpu.make_async_copy(tok_hbm.at[idx_v.at[k]], buf, sem).wait()
        p0 = lax.rem(k * CHUNK, T)

        @pl.loop(0, CHUNK)
        def _add(r):
            for j in range(D // LANES):
                s = pl.ds(j * LANES, LANES)
                buf[r, s] = buf[r, s] + pos2_v[p0 + r, s]

        pltpu.sync_copy(buf, out_hbm.at[pl.ds(row_base + k * CHUNK, CHUNK), :])

    start(0, buf0, sem0)
    start(1, buf1, sem1)

    @pl.loop(0, NCHUNK // 2 - 1)
    def _main(i):
        k = i * 2
        finish(k, buf0, sem0)
        start(k + 2, buf0, sem0)
        finish(k + 1, buf1, sem1)
        start(k + 3, buf1, sem1)

    finish(NCHUNK - 2, buf0, sem0)
    finish(NCHUNK - 1, buf1, sem1)


@functools.partial(jax.jit, static_argnames=())
def kernel(x, token_table, pos_table):
    x2 = x.astype(jnp.int32).reshape(LOOK // CHUNK, CHUNK)
    f = pl.kernel(
        _body,
        out_type=jax.ShapeDtypeStruct((LOOK, D), jnp.float32),
        mesh=plsc.VectorSubcoreMesh(core_axis_name="c", subcore_axis_name="s"),
        compiler_params=pltpu.CompilerParams(use_tc_tiling_on_sc=False),
        scratch_types=[
            pltpu.VMEM((NCHUNK, CHUNK), jnp.int32),
            pltpu.VMEM((2 * T, D), jnp.float32),
            pltpu.VMEM((CHUNK, D), jnp.float32),
            pltpu.VMEM((CHUNK, D), jnp.float32),
            pltpu.SemaphoreType.DMA,
            pltpu.SemaphoreType.DMA,
        ],
    )
    out = f(x2, token_table, pos_table)
    return out.reshape(B, T, D)


# speedup vs baseline: 4.1837x; 1.6323x over previous
import functools

import jax
import jax.numpy as jnp
from jax import lax
from jax.experimental import pallas as pl
from jax.experimental.pallas import tpu as pltpu
from jax.experimental.pallas import tpu_sc as plsc

T = 200
D = 64
B = 4096

NC = 2
NS = 16
NW = NC * NS
LOOK = B * T
PER_W = LOOK // NW
CHUNK = 128
NCHUNK = PER_W // CHUNK
LANES = 16
NBUF = 4


def _body(x_hbm, tok_hbm, pos_hbm, out_hbm, idx_v, pos2_v,
          b0, b1, b2, b3, g0, g1, g2, g3, s0, s1, s2, s3):
    bufs = (b0, b1, b2, b3)
    gsems = (g0, g1, g2, g3)
    ssems = (s0, s1, s2, s3)

    wid = lax.axis_index("s") * NC + lax.axis_index("c")
    base_chunk = wid * NCHUNK
    row_base = wid * PER_W

    pltpu.sync_copy(x_hbm.at[pl.ds(base_chunk, NCHUNK), :], idx_v)
    pltpu.sync_copy(pos_hbm, pos2_v.at[pl.ds(0, T), :])
    pltpu.sync_copy(pos_hbm, pos2_v.at[pl.ds(T, T), :])

    def g_start(k, j):
        pltpu.make_async_copy(tok_hbm.at[idx_v.at[k]], bufs[j], gsems[j]).start()

    def g_wait(k, j):
        pltpu.make_async_copy(tok_hbm.at[idx_v.at[k]], bufs[j], gsems[j]).wait()

    def s_copy(k, j):
        return pltpu.make_async_copy(
            bufs[j], out_hbm.at[pl.ds(row_base + k * CHUNK, CHUNK), :], ssems[j])

    def process(k, j):
        g_wait(k, j)
        p0 = lax.rem(k * CHUNK, T)
        buf = bufs[j]

        @plsc.parallel_loop(0, CHUNK, unroll=2)
        def _add(r):
            for jj in range(D // LANES):
                s = pl.ds(jj * LANES, LANES)
                buf[r, s] = buf[r, s] + pos2_v[p0 + r, s]

        s_copy(k, j).start()

    g_start(0, 0)
    g_start(1, 1)
    g_start(2, 2)
    process(0, 0)
    g_start(3, 3)
    process(1, 1)

    @pl.loop(0, (NCHUNK - NBUF) // NBUF)
    def _main(i):
        k = 2 + i * NBUF
        for jj in range(NBUF):
            kk = k + jj
            j_cur = (2 + jj) % NBUF
            j_pre = (j_cur + 2) % NBUF
            s_copy(kk - 2, j_pre).wait()
            g_start(kk + 2, j_pre)
            process(kk, j_cur)

    process(NCHUNK - 2, 2)
    process(NCHUNK - 1, 3)
    for kk, j in ((196, 0), (197, 1), (198, 2), (199, 3)):
        s_copy(kk, j).wait()


@functools.partial(jax.jit, static_argnames=())
def kernel(x, token_table, pos_table):
    x2 = x.astype(jnp.int32).reshape(LOOK // CHUNK, CHUNK)
    f = pl.kernel(
        _body,
        out_type=jax.ShapeDtypeStruct((LOOK, D), jnp.float32),
        mesh=plsc.VectorSubcoreMesh(core_axis_name="c", subcore_axis_name="s"),
        compiler_params=pltpu.CompilerParams(use_tc_tiling_on_sc=False),
        scratch_types=[
            pltpu.VMEM((NCHUNK, CHUNK), jnp.int32),
            pltpu.VMEM((2 * T, D), jnp.float32),
        ] + [pltpu.VMEM((CHUNK, D), jnp.float32)] * NBUF
          + [pltpu.SemaphoreType.DMA] * (2 * NBUF),
    )
    out = f(x2, token_table, pos_table)
    return out.reshape(B, T, D)


# speedup vs baseline: 5.4957x; 1.3136x over previous
import functools

import jax
import jax.numpy as jnp
from jax import lax
from jax.experimental import pallas as pl
from jax.experimental.pallas import tpu as pltpu
from jax.experimental.pallas import tpu_sc as plsc

T = 200
D = 64
B = 4096
V = 100000

NC = 2
NS = 16
NW = NC * NS
LOOK = B * T
PER_W = LOOK // NW
CHUNK = 64
NCHUNK = PER_W // CHUNK
LANES = 16
NBUF = 4
IDXROWS = PER_W // 128


def _body(x_hbm, tok_hbm, pos_hbm, out_hbm, idx_v, pos2_v,
          g0, g1, g2, g3, c0, c1, c2, c3, gs0, gs1, gs2, gs3,
          ss0, ss1, ss2, ss3):
    gbufs = (g0, g1, g2, g3)
    cbufs = (c0, c1, c2, c3)
    gsems = (gs0, gs1, gs2, gs3)
    ssems = (ss0, ss1, ss2, ss3)

    wid = lax.axis_index("s") * NC + lax.axis_index("c")
    row_base = wid * PER_W

    pltpu.sync_copy(x_hbm.at[pl.ds(wid * IDXROWS, IDXROWS), :], idx_v)
    pltpu.sync_copy(pos_hbm, pos2_v.at[pl.ds(0, T), :])
    pltpu.sync_copy(pos_hbm.at[pl.ds(0, CHUNK), :], pos2_v.at[pl.ds(T, CHUNK), :])

    def idx_slice(k, half):
        return idx_v.at[k // 2, pl.ds(half * CHUNK, CHUNK)]

    def g_start(k, half, j):
        pltpu.make_async_copy(
            tok_hbm.at[idx_slice(k, half)], gbufs[j], gsems[j]).start()

    def g_wait(k, half, j):
        pltpu.make_async_copy(
            tok_hbm.at[idx_slice(k, half)], gbufs[j], gsems[j]).wait()

    def s_copy(k, j):
        return pltpu.make_async_copy(
            cbufs[j], out_hbm.at[pl.ds(row_base + k * CHUNK, CHUNK), :],
            ssems[j])

    def process(k, half, j):
        g_wait(k, half, j)
        p0 = lax.rem(k * CHUNK, T)
        gbuf = gbufs[j]
        cbuf = cbufs[j]

        @plsc.parallel_loop(0, CHUNK, unroll=2)
        def _add(r):
            for jj in range(D // LANES):
                s = pl.ds(jj * LANES, LANES)
                cbuf[r, s] = gbuf[r, s] + pos2_v[p0 + r, s]

        s_copy(k, j).start()

    g_start(0, 0, 0)
    g_start(1, 1, 1)
    g_start(2, 0, 2)
    process(0, 0, 0)
    g_start(3, 1, 3)
    process(1, 1, 1)

    @pl.loop(0, (NCHUNK - NBUF) // NBUF)
    def _main(i):
        k = 2 + i * NBUF
        for jj in range(NBUF):
            kk = k + jj
            half = jj % 2
            j_cur = (2 + jj) % NBUF
            j_pre = (j_cur + 2) % NBUF
            s_copy(kk - 2, j_pre).wait()
            g_start(kk + 2, half, j_pre)
            process(kk, half, j_cur)

    process(NCHUNK - 2, 0, 2)
    process(NCHUNK - 1, 1, 3)
    for kk, j in ((NCHUNK - 4, 0), (NCHUNK - 3, 1),
                  (NCHUNK - 2, 2), (NCHUNK - 1, 3)):
        s_copy(kk, j).wait()


@functools.partial(jax.jit, static_argnames=())
def kernel(x, token_table, pos_table):
    x2 = x.astype(jnp.int32).reshape(LOOK // 128, 128)
    tok_p = jnp.pad(token_table, ((0, 0), (0, 128 - D)))
    f = pl.kernel(
        _body,
        out_type=jax.ShapeDtypeStruct((LOOK, D), jnp.float32),
        mesh=plsc.VectorSubcoreMesh(core_axis_name="c", subcore_axis_name="s"),
        compiler_params=pltpu.CompilerParams(use_tc_tiling_on_sc=True),
        scratch_types=[
            pltpu.VMEM((IDXROWS, 128), jnp.int32),
            pltpu.VMEM((T + CHUNK, D), jnp.float32),
        ] + [pltpu.VMEM((CHUNK, 128), jnp.float32)] * NBUF
          + [pltpu.VMEM((CHUNK, D), jnp.float32)] * NBUF
          + [pltpu.SemaphoreType.DMA] * (2 * NBUF),
    )
    out = f(x2, tok_p, pos_table)
    return out.reshape(B, T, D)
